# baseline (speedup 1.0000x reference)
import functools

import jax
import jax.numpy as jnp
from jax import lax
from jax.experimental import pallas as pl
from jax.experimental.pallas import tpu as pltpu
from jax.experimental.pallas import tpu_sc as plsc

N = 10000
E = 160000
D_IN, D_H, D_OUT = 256, 512, 128
NCHUNK = 4
FC = D_H // NCHUNK

BM = 2000
NBLK = N // BM

NSUB = 16
EB = 128
NBUF = 2
NB = 80
GB = 16
EPW = NB * EB
E_PAD = EPW * NSUB
ACC_ROWS = 10240
ZROWS = ACC_ROWS // NSUB
CPR = 624
CPR_REM = N - CPR * NSUB

EPS_BN = 1e-5
EPS_L2 = 1e-12


def _bn_coeffs(s, q, g, be):
    m = s / N
    v = q / N - m * m
    a1 = g * lax.rsqrt(v + EPS_BN)
    a0 = be - m * a1
    return a1, a0


def _stats_update(z_blk, so_ref, qo_ref):
    i = pl.program_id(0)

    @pl.when(i == 0)
    def _():
        so_ref[...] = jnp.zeros_like(so_ref)
        qo_ref[...] = jnp.zeros_like(qo_ref)

    so_ref[...] += jnp.sum(z_blk, axis=0, keepdims=True)
    qo_ref[...] += jnp.sum(z_blk * z_blk, axis=0, keepdims=True)


def _mlp_stage(u, W, b, stats=None, gb=None, post="stats"):
    din, dout = W.shape
    n_in = 3 + (4 if stats is not None else 0)

    def body(*refs):
        u_ref, w_ref, b_ref = refs[0], refs[1], refs[2]
        uu = u_ref[...]
        if stats is not None:
            s_ref, q_ref, g_ref, be_ref = refs[3:7]
            a1, a0 = _bn_coeffs(s_ref[...], q_ref[...], g_ref[...], be_ref[...])
            uu = jnp.maximum(uu * a1 + a0, 0.0)
        z = jnp.dot(uu.astype(jnp.bfloat16), w_ref[...],
                    preferred_element_type=jnp.float32) + b_ref[...]
        if post == "logsoftmax":
            z = z - jnp.max(z, axis=-1, keepdims=True)
            z = z - jnp.log(jnp.sum(jnp.exp(z), axis=-1, keepdims=True))
            refs[n_in][...] = z
        elif post == "plain":
            refs[n_in][...] = z
        else:
            refs[n_in][...] = z
            _stats_update(z, refs[n_in + 1], refs[n_in + 2])

    vspec = lambda d: pl.BlockSpec((1, d), lambda i: (0, 0))
    in_specs = [
        pl.BlockSpec((BM, din), lambda i: (i, 0)),
        pl.BlockSpec((din, dout), lambda i: (0, 0)),
        vspec(dout),
    ]
    args = [u, W, b]
    if stats is not None:
        in_specs += [vspec(din)] * 4
        args += [stats[0], stats[1], gb[0], gb[1]]

    if post in ("logsoftmax", "plain"):
        out_shape = jax.ShapeDtypeStruct((N, dout), jnp.float32)
        out_specs = pl.BlockSpec((BM, dout), lambda i: (i, 0))
    else:
        out_shape = (
            jax.ShapeDtypeStruct((N, dout), jnp.float32),
            jax.ShapeDtypeStruct((1, dout), jnp.float32),
            jax.ShapeDtypeStruct((1, dout), jnp.float32),
        )
        out_specs = (
            pl.BlockSpec((BM, dout), lambda i: (i, 0)),
            vspec(dout),
            vspec(dout),
        )
    return pl.pallas_call(
        body,
        grid=(NBLK,),
        in_specs=in_specs,
        out_specs=out_specs,
        out_shape=out_shape,
    )(*args)


def _bn_relu(u, s, q, g, be):
    d = u.shape[1]

    def body(u_ref, s_ref, q_ref, g_ref, be_ref, o_ref):
        a1, a0 = _bn_coeffs(s_ref[...], q_ref[...], g_ref[...], be_ref[...])
        o_ref[...] = jnp.maximum(u_ref[...] * a1 + a0, 0.0)

    vspec = lambda: pl.BlockSpec((1, d), lambda i: (0, 0))
    return pl.pallas_call(
        body,
        grid=(NBLK,),
        in_specs=[pl.BlockSpec((BM, d), lambda i: (i, 0)),
                  vspec(), vspec(), vspec(), vspec()],
        out_specs=pl.BlockSpec((BM, d), lambda i: (i, 0)),
        out_shape=jax.ShapeDtypeStruct((N, d), jnp.float32),
    )(u, s, q, g, be)


def _sage_combine(agg, hw, Wl):
    def body(agg_ref, hw_ref, wl_ref, y_ref, so_ref, qo_ref):
        acc = hw_ref[...]
        for k in range(NCHUNK):
            acc += jnp.dot(agg_ref[k].astype(jnp.bfloat16),
                           wl_ref[k * FC:(k + 1) * FC, :],
                           preferred_element_type=jnp.float32)
        nrm = jnp.maximum(jnp.sqrt(jnp.sum(acc * acc, axis=-1, keepdims=True)),
                          EPS_L2)
        y = acc / nrm
        y_ref[...] = y
        _stats_update(y, so_ref, qo_ref)

    vspec = lambda: pl.BlockSpec((1, D_H), lambda i: (0, 0))
    return pl.pallas_call(
        body,
        grid=(NBLK,),
        in_specs=[
            pl.BlockSpec((NCHUNK, BM, FC), lambda i: (0, i, 0)),
            pl.BlockSpec((BM, D_H), lambda i: (i, 0)),
            pl.BlockSpec((D_H, D_H), lambda i: (0, 0)),
        ],
        out_specs=(pl.BlockSpec((BM, D_H), lambda i: (i, 0)), vspec(), vspec()),
        out_shape=(
            jax.ShapeDtypeStruct((N, D_H), jnp.float32),
            jax.ShapeDtypeStruct((1, D_H), jnp.float32),
            jax.ShapeDtypeStruct((1, D_H), jnp.float32),
        ),
    )(agg, hw, Wl)


def _segment_sum_sc(h2d, gidx, dsti, zeros):
    mesh = plsc.VectorSubcoreMesh(core_axis_name="c", subcore_axis_name="s")

    @functools.partial(
        pl.kernel,
        out_type=jax.ShapeDtypeStruct((NCHUNK, N, FC), jnp.float32),
        mesh=mesh,
        scratch_types=[
            pltpu.VMEM((GB, EB), jnp.int32),
            pltpu.VMEM((GB, EB), jnp.int32),
            pltpu.VMEM((NBUF, EB, FC), jnp.float32),
            pltpu.VMEM_SHARED((ACC_ROWS, FC), jnp.float32),
            [pltpu.SemaphoreType.DMA] * NBUF,
        ],
    )
    def k(h_hbm, gidx_hbm, dst_hbm, zeros_hbm, out_hbm,
          gi_v, di_v, rows_v, acc_sh, sems):
        s = lax.axis_index("s")
        c = lax.axis_index("c")

        def do_chunk(kc):
            pltpu.sync_copy(zeros_hbm, acc_sh.at[pl.ds(s * ZROWS, ZROWS)])
            plsc.subcore_barrier()

            @pl.loop(0, NB, step=GB)
            def _(g):
                pltpu.sync_copy(gidx_hbm.at[kc].at[s].at[pl.ds(g, GB)], gi_v)
                pltpu.sync_copy(dst_hbm.at[s].at[pl.ds(g, GB)], di_v)

                @pl.loop(0, GB, step=NBUF)
                def _(b):
                    hs = [pltpu.async_copy(h_hbm.at[gi_v.at[b + j]],
                                           rows_v.at[j], sems[j])
                          for j in range(NBUF)]
                    for j in range(NBUF):
                        hs[j].wait()
                        pltpu.sync_copy(rows_v.at[j],
                                        acc_sh.at[di_v.at[b + j]], add=True)

            plsc.subcore_barrier()
            pltpu.sync_copy(acc_sh.at[pl.ds(s * CPR, CPR)],
                            out_hbm.at[kc].at[pl.ds(s * CPR, CPR)])

            @pl.when(s == 0)
            def _():
                pltpu.sync_copy(acc_sh.at[pl.ds(CPR * NSUB, CPR_REM)],
                                out_hbm.at[kc].at[pl.ds(CPR * NSUB, CPR_REM)])

            plsc.subcore_barrier()

        @pl.when(c == 0)
        def _():
            do_chunk(0)
            do_chunk(1)

        @pl.when(c == 1)
        def _():
            do_chunk(2)
            do_chunk(3)

    return k(h2d, gidx, dsti, zeros)


def kernel(x, edge_index, W_pre0, b_pre0, g_pre0, be_pre0, W_pre1, b_pre1,
           g_bn1, be_bn1, Wl1, bl1, Wr1, g_n1, be_n1, Wl2, bl2, Wr2,
           g_n2, be_n2, W_jk, b_jk, g_bn2, be_bn2, Wp0, bp0, g_p0, be_p0,
           Wp1, bp1):
    src = edge_index[0].astype(jnp.int32)
    dst = edge_index[1].astype(jnp.int32)
    pad = E_PAD - E
    srcp = jnp.concatenate([src, jnp.zeros((pad,), jnp.int32)])
    dstp = jnp.concatenate([dst, jnp.full((pad,), N, jnp.int32)])
    gidx = (srcp[None, :] * NCHUNK
            + jnp.arange(NCHUNK, dtype=jnp.int32)[:, None])
    gidx = gidx.reshape(NCHUNK, NSUB, NB, EB)
    dsti = dstp.reshape(NSUB, NB, EB)
    zeros = jnp.zeros((ZROWS, FC), jnp.float32)
    r = lambda v: v.reshape(1, -1)
    bf = lambda w: w.astype(jnp.bfloat16)

    z1, s1, q1 = _mlp_stage(x, bf(W_pre0), r(b_pre0))
    z2, s2, q2 = _mlp_stage(z1, bf(W_pre1), r(b_pre1), stats=(s1, q1),
                            gb=(r(g_pre0), r(be_pre0)))
    h = _bn_relu(z2, s2, q2, r(g_bn1), r(be_bn1))

    agg1 = _segment_sum_sc(h.reshape(N * NCHUNK, FC), gidx, dsti, zeros)
    hw1 = _mlp_stage(h, bf(Wr1), r(bl1), post="plain")
    y1, sy1, qy1 = _sage_combine(agg1, hw1, bf(Wl1))
    h1 = _bn_relu(y1, sy1, qy1, r(g_n1), r(be_n1))

    agg2 = _segment_sum_sc(h1.reshape(N * NCHUNK, FC), gidx, dsti, zeros)
    hw2 = _mlp_stage(h1, bf(Wr2), r(bl2), post="plain")
    y2, sy2, qy2 = _sage_combine(agg2, hw2, bf(Wl2))

    z3, s3, q3 = _mlp_stage(y2, bf(W_jk), r(b_jk), stats=(sy2, qy2),
                            gb=(r(g_n2), r(be_n2)))
    z4, s4, q4 = _mlp_stage(z3, bf(Wp0), r(bp0), stats=(s3, q3),
                            gb=(r(g_bn2), r(be_bn2)))
    out = _mlp_stage(z4, bf(Wp1), r(bp1), stats=(s4, q4),
                     gb=(r(g_p0), r(be_p0)), post="logsoftmax")
    return out


# speedup vs baseline: 1.0782x; 1.0782x over previous
import functools

import jax
import jax.numpy as jnp
from jax import lax
from jax.experimental import pallas as pl
from jax.experimental.pallas import tpu as pltpu
from jax.experimental.pallas import tpu_sc as plsc

N = 10000
E = 160000
D_IN, D_H, D_OUT = 256, 512, 128
NCHUNK = 4
FC = D_H // NCHUNK

BM = 2000
NBLK = N // BM

NSUB = 16
EB = 128
NBUF = 2
NB = 80
GB = 16
EPW = NB * EB
E_PAD = EPW * NSUB
ACC_ROWS = 10240
ZROWS = ACC_ROWS // NSUB
CPR = 624
CPR_REM = N - CPR * NSUB

EPS_BN = 1e-5
EPS_L2 = 1e-12


def _bn_coeffs(s, q, g, be):
    m = s / N
    v = q / N - m * m
    a1 = g * lax.rsqrt(v + EPS_BN)
    a0 = be - m * a1
    return a1, a0


def _stats_update(z_blk, so_ref, qo_ref):
    i = pl.program_id(0)

    @pl.when(i == 0)
    def _():
        so_ref[...] = jnp.zeros_like(so_ref)
        qo_ref[...] = jnp.zeros_like(qo_ref)

    so_ref[...] += jnp.sum(z_blk, axis=0, keepdims=True)
    qo_ref[...] += jnp.sum(z_blk * z_blk, axis=0, keepdims=True)


def _mlp_stage(u, W, b, stats=None, gb=None, post="stats"):
    din, dout = W.shape
    n_in = 3 + (4 if stats is not None else 0)

    def body(*refs):
        u_ref, w_ref, b_ref = refs[0], refs[1], refs[2]
        uu = u_ref[...]
        if stats is not None:
            s_ref, q_ref, g_ref, be_ref = refs[3:7]
            a1, a0 = _bn_coeffs(s_ref[...], q_ref[...], g_ref[...], be_ref[...])
            uu = jnp.maximum(uu * a1 + a0, 0.0)
        z = jnp.dot(uu.astype(jnp.bfloat16), w_ref[...],
                    preferred_element_type=jnp.float32) + b_ref[...]
        if post == "logsoftmax":
            z = z - jnp.max(z, axis=-1, keepdims=True)
            z = z - jnp.log(jnp.sum(jnp.exp(z), axis=-1, keepdims=True))
            refs[n_in][...] = z
        elif post == "plain":
            refs[n_in][...] = z
        else:
            refs[n_in][...] = z
            _stats_update(z, refs[n_in + 1], refs[n_in + 2])

    vspec = lambda d: pl.BlockSpec((1, d), lambda i: (0, 0))
    in_specs = [
        pl.BlockSpec((BM, din), lambda i: (i, 0)),
        pl.BlockSpec((din, dout), lambda i: (0, 0)),
        vspec(dout),
    ]
    args = [u, W, b]
    if stats is not None:
        in_specs += [vspec(din)] * 4
        args += [stats[0], stats[1], gb[0], gb[1]]

    if post in ("logsoftmax", "plain"):
        out_shape = jax.ShapeDtypeStruct((N, dout), jnp.float32)
        out_specs = pl.BlockSpec((BM, dout), lambda i: (i, 0))
    else:
        out_shape = (
            jax.ShapeDtypeStruct((N, dout), jnp.float32),
            jax.ShapeDtypeStruct((1, dout), jnp.float32),
            jax.ShapeDtypeStruct((1, dout), jnp.float32),
        )
        out_specs = (
            pl.BlockSpec((BM, dout), lambda i: (i, 0)),
            vspec(dout),
            vspec(dout),
        )
    return pl.pallas_call(
        body,
        grid=(NBLK,),
        in_specs=in_specs,
        out_specs=out_specs,
        out_shape=out_shape,
    )(*args)


def _bn_inline(z, g, be):
    m = jnp.mean(z, axis=0, keepdims=True)
    v = jnp.mean(z * z, axis=0, keepdims=True) - m * m
    return g * (z - m) * lax.rsqrt(v + EPS_BN) + be


def _pre_mlp(x, W0, b0, g0, be0, W1, b1, g1, be1):
    def body(x_ref, w0_ref, b0_ref, g0_ref, be0_ref, w1_ref, b1_ref,
             g1_ref, be1_ref, h_ref):
        z = jnp.dot(x_ref[...].astype(jnp.bfloat16), w0_ref[...],
                    preferred_element_type=jnp.float32) + b0_ref[...]
        z = jnp.maximum(_bn_inline(z, g0_ref[...], be0_ref[...]), 0.0)
        z = jnp.dot(z.astype(jnp.bfloat16), w1_ref[...],
                    preferred_element_type=jnp.float32) + b1_ref[...]
        h_ref[...] = jnp.maximum(_bn_inline(z, g1_ref[...], be1_ref[...]), 0.0)

    return pl.pallas_call(
        body,
        out_shape=jax.ShapeDtypeStruct((N, D_H), jnp.float32),
    )(x, W0, b0, g0, be0, W1, b1, g1, be1)


def _post_mlp(y2, sy2, qy2, g_n2, be_n2, W_jk, b_jk, g_bn2, be_bn2,
              Wp0, bp0, g_p0, be_p0, Wp1, bp1):
    def body(y_ref, s_ref, q_ref, gn_ref, ben_ref, wj_ref, bj_ref,
             gb_ref, beb_ref, w0_ref, b0_ref, gp_ref, bep_ref,
             w1_ref, b1_ref, o_ref):
        a1, a0 = _bn_coeffs(s_ref[...], q_ref[...], gn_ref[...], ben_ref[...])
        h2 = jnp.maximum(y_ref[...] * a1 + a0, 0.0)
        z = jnp.dot(h2.astype(jnp.bfloat16), wj_ref[...],
                    preferred_element_type=jnp.float32) + bj_ref[...]
        z = jnp.maximum(_bn_inline(z, gb_ref[...], beb_ref[...]), 0.0)
        z = jnp.dot(z.astype(jnp.bfloat16), w0_ref[...],
                    preferred_element_type=jnp.float32) + b0_ref[...]
        z = jnp.maximum(_bn_inline(z, gp_ref[...], bep_ref[...]), 0.0)
        z = jnp.dot(z.astype(jnp.bfloat16), w1_ref[...],
                    preferred_element_type=jnp.float32) + b1_ref[...]
        z = z - jnp.max(z, axis=-1, keepdims=True)
        o_ref[...] = z - jnp.log(jnp.sum(jnp.exp(z), axis=-1, keepdims=True))

    return pl.pallas_call(
        body,
        out_shape=jax.ShapeDtypeStruct((N, D_OUT), jnp.float32),
    )(y2, sy2, qy2, g_n2, be_n2, W_jk, b_jk, g_bn2, be_bn2,
      Wp0, bp0, g_p0, be_p0, Wp1, bp1)


def _bn_relu(u, s, q, g, be):
    d = u.shape[1]

    def body(u_ref, s_ref, q_ref, g_ref, be_ref, o_ref):
        a1, a0 = _bn_coeffs(s_ref[...], q_ref[...], g_ref[...], be_ref[...])
        o_ref[...] = jnp.maximum(u_ref[...] * a1 + a0, 0.0)

    vspec = lambda: pl.BlockSpec((1, d), lambda i: (0, 0))
    return pl.pallas_call(
        body,
        grid=(NBLK,),
        in_specs=[pl.BlockSpec((BM, d), lambda i: (i, 0)),
                  vspec(), vspec(), vspec(), vspec()],
        out_specs=pl.BlockSpec((BM, d), lambda i: (i, 0)),
        out_shape=jax.ShapeDtypeStruct((N, d), jnp.float32),
    )(u, s, q, g, be)


def _sage_combine(agg, hw, Wl):
    def body(agg_ref, hw_ref, wl_ref, y_ref, so_ref, qo_ref):
        acc = hw_ref[...]
        for k in range(NCHUNK):
            acc += jnp.dot(agg_ref[k].astype(jnp.bfloat16),
                           wl_ref[k * FC:(k + 1) * FC, :],
                           preferred_element_type=jnp.float32)
        nrm = jnp.maximum(jnp.sqrt(jnp.sum(acc * acc, axis=-1, keepdims=True)),
                          EPS_L2)
        y = acc / nrm
        y_ref[...] = y
        _stats_update(y, so_ref, qo_ref)

    vspec = lambda: pl.BlockSpec((1, D_H), lambda i: (0, 0))
    return pl.pallas_call(
        body,
        grid=(NBLK,),
        in_specs=[
            pl.BlockSpec((NCHUNK, BM, FC), lambda i: (0, i, 0)),
            pl.BlockSpec((BM, D_H), lambda i: (i, 0)),
            pl.BlockSpec((D_H, D_H), lambda i: (0, 0)),
        ],
        out_specs=(pl.BlockSpec((BM, D_H), lambda i: (i, 0)), vspec(), vspec()),
        out_shape=(
            jax.ShapeDtypeStruct((N, D_H), jnp.float32),
            jax.ShapeDtypeStruct((1, D_H), jnp.float32),
            jax.ShapeDtypeStruct((1, D_H), jnp.float32),
        ),
    )(agg, hw, Wl)


def _segment_sum_sc(h2d, gidx, dsti, zeros):
    mesh = plsc.VectorSubcoreMesh(core_axis_name="c", subcore_axis_name="s")

    @functools.partial(
        pl.kernel,
        out_type=jax.ShapeDtypeStruct((NCHUNK, N, FC), jnp.float32),
        mesh=mesh,
        scratch_types=[
            pltpu.VMEM((GB, EB), jnp.int32),
            pltpu.VMEM((GB, EB), jnp.int32),
            pltpu.VMEM((NBUF, EB, FC), jnp.float32),
            pltpu.VMEM_SHARED((ACC_ROWS, FC), jnp.float32),
            [pltpu.SemaphoreType.DMA] * NBUF,
        ],
    )
    def k(h_hbm, gidx_hbm, dst_hbm, zeros_hbm, out_hbm,
          gi_v, di_v, rows_v, acc_sh, sems):
        s = lax.axis_index("s")
        c = lax.axis_index("c")

        def do_chunk(kc):
            pltpu.sync_copy(zeros_hbm, acc_sh.at[pl.ds(s * ZROWS, ZROWS)])
            plsc.subcore_barrier()

            @pl.loop(0, NB, step=GB)
            def _(g):
                pltpu.sync_copy(gidx_hbm.at[kc].at[s].at[pl.ds(g, GB)], gi_v)
                pltpu.sync_copy(dst_hbm.at[s].at[pl.ds(g, GB)], di_v)

                @pl.loop(0, GB, step=NBUF)
                def _(b):
                    hs = [pltpu.async_copy(h_hbm.at[gi_v.at[b + j]],
                                           rows_v.at[j], sems[j])
                          for j in range(NBUF)]
                    for j in range(NBUF):
                        hs[j].wait()
                        pltpu.sync_copy(rows_v.at[j],
                                        acc_sh.at[di_v.at[b + j]], add=True)

            plsc.subcore_barrier()
            pltpu.sync_copy(acc_sh.at[pl.ds(s * CPR, CPR)],
                            out_hbm.at[kc].at[pl.ds(s * CPR, CPR)])

            @pl.when(s == 0)
            def _():
                pltpu.sync_copy(acc_sh.at[pl.ds(CPR * NSUB, CPR_REM)],
                                out_hbm.at[kc].at[pl.ds(CPR * NSUB, CPR_REM)])

            plsc.subcore_barrier()

        @pl.when(c == 0)
        def _():
            do_chunk(0)
            do_chunk(1)

        @pl.when(c == 1)
        def _():
            do_chunk(2)
            do_chunk(3)

    return k(h2d, gidx, dsti, zeros)


def kernel(x, edge_index, W_pre0, b_pre0, g_pre0, be_pre0, W_pre1, b_pre1,
           g_bn1, be_bn1, Wl1, bl1, Wr1, g_n1, be_n1, Wl2, bl2, Wr2,
           g_n2, be_n2, W_jk, b_jk, g_bn2, be_bn2, Wp0, bp0, g_p0, be_p0,
           Wp1, bp1):
    src = edge_index[0].astype(jnp.int32)
    dst = edge_index[1].astype(jnp.int32)
    pad = E_PAD - E
    srcp = jnp.concatenate([src, jnp.zeros((pad,), jnp.int32)])
    dstp = jnp.concatenate([dst, jnp.full((pad,), N, jnp.int32)])
    gidx = (srcp[None, :] * NCHUNK
            + jnp.arange(NCHUNK, dtype=jnp.int32)[:, None])
    gidx = gidx.reshape(NCHUNK, NSUB, NB, EB)
    dsti = dstp.reshape(NSUB, NB, EB)
    zeros = jnp.zeros((ZROWS, FC), jnp.float32)
    r = lambda v: v.reshape(1, -1)
    bf = lambda w: w.astype(jnp.bfloat16)

    h = _pre_mlp(x, bf(W_pre0), r(b_pre0), r(g_pre0), r(be_pre0),
                 bf(W_pre1), r(b_pre1), r(g_bn1), r(be_bn1))

    agg1 = _segment_sum_sc(h.reshape(N * NCHUNK, FC), gidx, dsti, zeros)
    hw1 = _mlp_stage(h, bf(Wr1), r(bl1), post="plain")
    y1, sy1, qy1 = _sage_combine(agg1, hw1, bf(Wl1))
    h1 = _bn_relu(y1, sy1, qy1, r(g_n1), r(be_n1))

    agg2 = _segment_sum_sc(h1.reshape(N * NCHUNK, FC), gidx, dsti, zeros)
    hw2 = _mlp_stage(h1, bf(Wr2), r(bl2), post="plain")
    y2, sy2, qy2 = _sage_combine(agg2, hw2, bf(Wl2))

    out = _post_mlp(y2, sy2, qy2, r(g_n2), r(be_n2), bf(W_jk), r(b_jk),
                    r(g_bn2), r(be_bn2), bf(Wp0), r(bp0), r(g_p0), r(be_p0),
                    bf(Wp1), r(bp1))
    return out
